# baseline (speedup 1.0000x reference)
import functools

import jax
import jax.numpy as jnp
from jax import lax
from jax.experimental import pallas as pl
from jax.experimental.pallas import tpu as pltpu
from jax.experimental.pallas import tpu_sc as plsc

N = 10000
E = 320000
D = 128
R = 5

NC = 2
NS = 16
NW = NC * NS
CH = 128
FAST_C = 0
CPW_F = 80
CPW_S = 80
CPWMAX = CPW_F
E_PAD = NS * (CPW_F + CPW_S) * CH
ACC_ROWS = 10112
ZROWS = ACC_ROWS // NS
TRASH = ACC_ROWS - 1


NBUF = 2


def _sc_agg_body(table, gidx, dstx, zeros, out, idxv, dstr, rows, acc, *sems):
    gsem = sems[0:NBUF]
    ssem = sems[NBUF:2 * NBUF]
    dsem = sems[2 * NBUF:3 * NBUF]
    c = lax.axis_index("c")
    s = lax.axis_index("s")
    wid = s * NC + c
    cpw = jnp.where(c == FAST_C, CPW_F, CPW_S)
    pltpu.sync_copy(gidx.at[wid], idxv)
    pltpu.sync_copy(zeros, acc.at[pl.ds(s * ZROWS, ZROWS)])
    plsc.subcore_barrier()

    for b in range(NBUF):
        pltpu.async_copy(dstx.at[wid, b], dstr.at[b], dsem[b])
        pltpu.async_copy(table.at[idxv.at[b]], rows.at[b], gsem[b])

    def group(g, carry):
        for b in range(NBUF):
            j = g * NBUF + b
            pltpu.make_async_copy(table.at[idxv.at[j]], rows.at[b], gsem[b]).wait()
            pltpu.make_async_copy(dstx.at[wid, j], dstr.at[b], dsem[b]).wait()
            pltpu.async_copy(rows.at[b], acc.at[dstr.at[b]], ssem[b], add=True)
            jn = j + NBUF

            @pl.when(jn < cpw)
            def _():
                pltpu.make_async_copy(
                    rows.at[b], acc.at[dstr.at[b]], ssem[b]
                ).wait()
                pltpu.async_copy(dstx.at[wid, jn], dstr.at[b], dsem[b])
                pltpu.async_copy(table.at[idxv.at[jn]], rows.at[b], gsem[b])

        return carry

    lax.fori_loop(0, cpw // NBUF, group, 0)
    for b in range(NBUF):
        pltpu.make_async_copy(rows.at[b], acc.at[dstr.at[b]], ssem[b]).wait()
    plsc.subcore_barrier()
    pltpu.sync_copy(
        acc.at[pl.ds(s * ZROWS, ZROWS)], out.at[c, pl.ds(s * ZROWS, ZROWS)]
    )


@functools.cache
def _make_sc_agg():
    return pl.kernel(
        _sc_agg_body,
        out_type=jax.ShapeDtypeStruct((NC, ACC_ROWS, D), jnp.float32),
        mesh=plsc.VectorSubcoreMesh(
            core_axis_name="c", subcore_axis_name="s", num_cores=NC, num_subcores=NS
        ),
        scratch_types=[
            pltpu.VMEM((CPWMAX, CH), jnp.int32),
            pltpu.VMEM((NBUF, CH), jnp.int32),
            pltpu.VMEM((NBUF, CH, D), jnp.float32),
            pltpu.VMEM_SHARED((ACC_ROWS, D), jnp.float32),
        ] + [pltpu.SemaphoreType.DMA] * (3 * NBUF),
    )


def _sc_agg(table, gidx, dstx, zeros):
    return _make_sc_agg()(table, gidx, dstx, zeros)


BN = 5000


def _proj_body(wc_ref, x_ref, b_ref, lw_ref, bias_ref, xw_ref, lp_ref):
    x_blk = x_ref[...]
    b0 = b_ref[0]
    b1 = b_ref[1]
    for r in range(R):
        w_r = wc_ref[r, 0] * b0 + wc_ref[r, 1] * b1
        xw_ref[r] = jnp.dot(x_blk, w_r, preferred_element_type=jnp.float32)
    lp_ref[...] = (
        jnp.dot(x_blk, lw_ref[...], preferred_element_type=jnp.float32)
        + bias_ref[...]
    )


def _proj(x, bases, w_comp, loop_w, bias):
    return pl.pallas_call(
        _proj_body,
        grid=(N // BN,),
        in_specs=[
            pl.BlockSpec(memory_space=pltpu.SMEM),
            pl.BlockSpec((BN, D), lambda i: (i, 0)),
            pl.BlockSpec((2, D, D), lambda i: (0, 0, 0)),
            pl.BlockSpec((D, D), lambda i: (0, 0)),
            pl.BlockSpec((1, D), lambda i: (0, 0)),
        ],
        out_specs=[
            pl.BlockSpec((R, BN, D), lambda i: (0, i, 0)),
            pl.BlockSpec((BN, D), lambda i: (i, 0)),
        ],
        out_shape=[
            jax.ShapeDtypeStruct((R, N, D), jnp.float32),
            jax.ShapeDtypeStruct((N, D), jnp.float32),
        ],
    )(w_comp, x, bases, loop_w, bias.reshape(1, D))


def _proj_fused_body(wc_ref, p_ref, lp_ref, b_ref, lw_ref, bias_ref, xw_ref, lp2_ref):
    h = jnp.maximum(p_ref[0] + p_ref[1] + lp_ref[...], 0.0)
    b0 = b_ref[0]
    b1 = b_ref[1]
    for r in range(R):
        w_r = wc_ref[r, 0] * b0 + wc_ref[r, 1] * b1
        xw_ref[r] = jnp.dot(h, w_r, preferred_element_type=jnp.float32)
    lp2_ref[...] = (
        jnp.dot(h, lw_ref[...], preferred_element_type=jnp.float32)
        + bias_ref[...]
    )


def _proj_fused(part, lp, bases, w_comp, loop_w, bias):
    return pl.pallas_call(
        _proj_fused_body,
        grid=(N // BN,),
        in_specs=[
            pl.BlockSpec(memory_space=pltpu.SMEM),
            pl.BlockSpec((NC, BN, D), lambda i: (0, i, 0)),
            pl.BlockSpec((BN, D), lambda i: (i, 0)),
            pl.BlockSpec((2, D, D), lambda i: (0, 0, 0)),
            pl.BlockSpec((D, D), lambda i: (0, 0)),
            pl.BlockSpec((1, D), lambda i: (0, 0)),
        ],
        out_specs=[
            pl.BlockSpec((R, BN, D), lambda i: (0, i, 0)),
            pl.BlockSpec((BN, D), lambda i: (i, 0)),
        ],
        out_shape=[
            jax.ShapeDtypeStruct((R, N, D), jnp.float32),
            jax.ShapeDtypeStruct((N, D), jnp.float32),
        ],
    )(w_comp, part, lp, bases, loop_w, bias.reshape(1, D))


def _final_body(x_ref, p_ref, lp_ref, h2_ref, cat_ref):
    h2 = jnp.maximum(p_ref[0] + p_ref[1] + lp_ref[...], 0.0)
    h2_ref[...] = h2
    cat_ref[...] = jnp.concatenate([x_ref[...], h2], axis=1)


def _final(x, part, lp):
    return pl.pallas_call(
        _final_body,
        grid=(N // BN,),
        in_specs=[
            pl.BlockSpec((BN, D), lambda i: (i, 0)),
            pl.BlockSpec((NC, BN, D), lambda i: (0, i, 0)),
            pl.BlockSpec((BN, D), lambda i: (i, 0)),
        ],
        out_specs=[
            pl.BlockSpec((BN, D), lambda i: (i, 0)),
            pl.BlockSpec((BN, 2 * D), lambda i: (i, 0)),
        ],
        out_shape=[
            jax.ShapeDtypeStruct((N, D), jnp.float32),
            jax.ShapeDtypeStruct((N, 2 * D), jnp.float32),
        ],
    )(x, part, lp)


def _edge_layout(flat, fill):
    ef = NS * CPW_F * CH
    fast = flat[:ef].reshape(NS, CPW_F, CH)
    slow = flat[ef:].reshape(NS, CPW_S, CH)
    if CPW_F != CPW_S:
        slow = jnp.concatenate(
            [slow, jnp.full((NS, CPW_F - CPW_S, CH), fill, jnp.int32)], axis=1
        )
    per_c = [fast, slow] if FAST_C == 0 else [slow, fast]
    return jnp.stack(per_c, axis=1).reshape(NW, CPWMAX, CH)


def kernel(x, edge_index, etype, bases1, w_comp1, loop_w1, bias1,
           bases2, w_comp2, loop_w2, bias2):
    src = edge_index[0]
    dst = edge_index[1]
    pad = E_PAD - E
    pad_g = jnp.arange(pad, dtype=jnp.int32) % (R * N)
    pad_d = N + jnp.arange(pad, dtype=jnp.int32) % (ACC_ROWS - N)
    gidx = _edge_layout(jnp.concatenate([etype * N + src, pad_g]), 0)
    dstx = _edge_layout(jnp.concatenate([dst, pad_d]), TRASH)
    zeros = jnp.zeros((ZROWS, D), jnp.float32)

    xw1, lp1 = _proj(x, bases1, w_comp1, loop_w1, bias1)
    part1 = _sc_agg(xw1.reshape(R * N, D), gidx, dstx, zeros)
    xw2, lp2 = _proj_fused(part1, lp1, bases2, w_comp2, loop_w2, bias2)
    part2 = _sc_agg(xw2.reshape(R * N, D), gidx, dstx, zeros)
    h2, cat = _final(x, part2, lp2)
    return (x, h2, cat)


# speedup vs baseline: 1.0096x; 1.0096x over previous
import functools

import jax
import jax.numpy as jnp
from jax import lax
from jax.experimental import pallas as pl
from jax.experimental.pallas import tpu as pltpu
from jax.experimental.pallas import tpu_sc as plsc

N = 10000
E = 320000
D = 128
R = 5

NC = 2
NS = 16
NW = NC * NS
CH = 128
FAST_C = 0
CPW_F = 80
CPW_S = 80
CPWMAX = CPW_F
E_PAD = NS * (CPW_F + CPW_S) * CH
ACC_ROWS = 10112
ZROWS = ACC_ROWS // NS
TRASH = ACC_ROWS - 1


NBUF = 2


def _sc_agg_body(table, gidx, dstx, zeros, out, idxv, dstr, rows, acc, *sems):
    gsem = sems[0:NBUF]
    ssem = sems[NBUF:2 * NBUF]
    dsem = sems[2 * NBUF:3 * NBUF]
    c = lax.axis_index("c")
    s = lax.axis_index("s")
    wid = s * NC + c
    cpw = jnp.where(c == FAST_C, CPW_F, CPW_S)
    pltpu.sync_copy(gidx.at[wid], idxv)
    pltpu.sync_copy(zeros, acc.at[pl.ds(s * ZROWS, ZROWS)])
    plsc.subcore_barrier()

    for b in range(NBUF):
        pltpu.async_copy(dstx.at[wid, b], dstr.at[b], dsem[b])
        pltpu.async_copy(table.at[idxv.at[b]], rows.at[b], gsem[b])

    def group(g, carry):
        for b in range(NBUF):
            j = g * NBUF + b
            pltpu.make_async_copy(table.at[idxv.at[j]], rows.at[b], gsem[b]).wait()
            pltpu.make_async_copy(dstx.at[wid, j], dstr.at[b], dsem[b]).wait()
            pltpu.async_copy(rows.at[b], acc.at[dstr.at[b]], ssem[b], add=True)
            jn = j + NBUF

            @pl.when(jn < cpw)
            def _():
                pltpu.make_async_copy(
                    rows.at[b], acc.at[dstr.at[b]], ssem[b]
                ).wait()
                pltpu.async_copy(dstx.at[wid, jn], dstr.at[b], dsem[b])
                pltpu.async_copy(table.at[idxv.at[jn]], rows.at[b], gsem[b])

        return carry

    lax.fori_loop(0, cpw // NBUF, group, 0)
    for b in range(NBUF):
        pltpu.make_async_copy(rows.at[b], acc.at[dstr.at[b]], ssem[b]).wait()
    plsc.subcore_barrier()
    pltpu.sync_copy(
        acc.at[pl.ds(s * ZROWS, ZROWS)], out.at[c, pl.ds(s * ZROWS, ZROWS)]
    )


@functools.cache
def _make_sc_agg():
    return pl.kernel(
        _sc_agg_body,
        out_type=jax.ShapeDtypeStruct((NC, ACC_ROWS, D), jnp.float32),
        mesh=plsc.VectorSubcoreMesh(
            core_axis_name="c", subcore_axis_name="s", num_cores=NC, num_subcores=NS
        ),
        scratch_types=[
            pltpu.VMEM((CPWMAX, CH), jnp.int32),
            pltpu.VMEM((NBUF, CH), jnp.int32),
            pltpu.VMEM((NBUF, CH, D), jnp.float32),
            pltpu.VMEM_SHARED((ACC_ROWS, D), jnp.float32),
        ] + [pltpu.SemaphoreType.DMA] * (3 * NBUF),
    )


def _sc_agg(table, gidx, dstx, zeros):
    return _make_sc_agg()(table, gidx, dstx, zeros)


BN = 2000


def _proj_body(wc_ref, x_ref, b_ref, lw_ref, bias_ref, xw_ref, lp_ref):
    x_blk = x_ref[...]
    b0 = b_ref[0]
    b1 = b_ref[1]
    for r in range(R):
        w_r = wc_ref[r, 0] * b0 + wc_ref[r, 1] * b1
        xw_ref[r] = jnp.dot(x_blk, w_r, preferred_element_type=jnp.float32)
    lp_ref[...] = (
        jnp.dot(x_blk, lw_ref[...], preferred_element_type=jnp.float32)
        + bias_ref[...]
    )


def _proj(x, bases, w_comp, loop_w, bias):
    return pl.pallas_call(
        _proj_body,
        grid=(N // BN,),
        in_specs=[
            pl.BlockSpec(memory_space=pltpu.SMEM),
            pl.BlockSpec((BN, D), lambda i: (i, 0)),
            pl.BlockSpec((2, D, D), lambda i: (0, 0, 0)),
            pl.BlockSpec((D, D), lambda i: (0, 0)),
            pl.BlockSpec((1, D), lambda i: (0, 0)),
        ],
        out_specs=[
            pl.BlockSpec((R, BN, D), lambda i: (0, i, 0)),
            pl.BlockSpec((BN, D), lambda i: (i, 0)),
        ],
        out_shape=[
            jax.ShapeDtypeStruct((R, N, D), jnp.float32),
            jax.ShapeDtypeStruct((N, D), jnp.float32),
        ],
    )(w_comp, x, bases, loop_w, bias.reshape(1, D))


def _proj_fused_body(wc_ref, p_ref, lp_ref, b_ref, lw_ref, bias_ref, xw_ref, lp2_ref):
    h = jnp.maximum(p_ref[0] + p_ref[1] + lp_ref[...], 0.0)
    b0 = b_ref[0]
    b1 = b_ref[1]
    for r in range(R):
        w_r = wc_ref[r, 0] * b0 + wc_ref[r, 1] * b1
        xw_ref[r] = jnp.dot(h, w_r, preferred_element_type=jnp.float32)
    lp2_ref[...] = (
        jnp.dot(h, lw_ref[...], preferred_element_type=jnp.float32)
        + bias_ref[...]
    )


def _proj_fused(part, lp, bases, w_comp, loop_w, bias):
    return pl.pallas_call(
        _proj_fused_body,
        grid=(N // BN,),
        in_specs=[
            pl.BlockSpec(memory_space=pltpu.SMEM),
            pl.BlockSpec((NC, BN, D), lambda i: (0, i, 0)),
            pl.BlockSpec((BN, D), lambda i: (i, 0)),
            pl.BlockSpec((2, D, D), lambda i: (0, 0, 0)),
            pl.BlockSpec((D, D), lambda i: (0, 0)),
            pl.BlockSpec((1, D), lambda i: (0, 0)),
        ],
        out_specs=[
            pl.BlockSpec((R, BN, D), lambda i: (0, i, 0)),
            pl.BlockSpec((BN, D), lambda i: (i, 0)),
        ],
        out_shape=[
            jax.ShapeDtypeStruct((R, N, D), jnp.float32),
            jax.ShapeDtypeStruct((N, D), jnp.float32),
        ],
    )(w_comp, part, lp, bases, loop_w, bias.reshape(1, D))


def _final_body(x_ref, p_ref, lp_ref, h2_ref, cat_ref):
    h2 = jnp.maximum(p_ref[0] + p_ref[1] + lp_ref[...], 0.0)
    h2_ref[...] = h2
    cat_ref[...] = jnp.concatenate([x_ref[...], h2], axis=1)


def _final(x, part, lp):
    return pl.pallas_call(
        _final_body,
        grid=(N // BN,),
        in_specs=[
            pl.BlockSpec((BN, D), lambda i: (i, 0)),
            pl.BlockSpec((NC, BN, D), lambda i: (0, i, 0)),
            pl.BlockSpec((BN, D), lambda i: (i, 0)),
        ],
        out_specs=[
            pl.BlockSpec((BN, D), lambda i: (i, 0)),
            pl.BlockSpec((BN, 2 * D), lambda i: (i, 0)),
        ],
        out_shape=[
            jax.ShapeDtypeStruct((N, D), jnp.float32),
            jax.ShapeDtypeStruct((N, 2 * D), jnp.float32),
        ],
    )(x, part, lp)


def _edge_layout(flat, fill):
    ef = NS * CPW_F * CH
    fast = flat[:ef].reshape(NS, CPW_F, CH)
    slow = flat[ef:].reshape(NS, CPW_S, CH)
    if CPW_F != CPW_S:
        slow = jnp.concatenate(
            [slow, jnp.full((NS, CPW_F - CPW_S, CH), fill, jnp.int32)], axis=1
        )
    per_c = [fast, slow] if FAST_C == 0 else [slow, fast]
    return jnp.stack(per_c, axis=1).reshape(NW, CPWMAX, CH)


def kernel(x, edge_index, etype, bases1, w_comp1, loop_w1, bias1,
           bases2, w_comp2, loop_w2, bias2):
    src = edge_index[0]
    dst = edge_index[1]
    pad = E_PAD - E
    pad_g = jnp.arange(pad, dtype=jnp.int32) % (R * N)
    pad_d = N + jnp.arange(pad, dtype=jnp.int32) % (ACC_ROWS - N)
    gidx = _edge_layout(jnp.concatenate([etype * N + src, pad_g]), 0)
    dstx = _edge_layout(jnp.concatenate([dst, pad_d]), TRASH)
    zeros = jnp.zeros((ZROWS, D), jnp.float32)

    xw1, lp1 = _proj(x, bases1, w_comp1, loop_w1, bias1)
    part1 = _sc_agg(xw1.reshape(R * N, D), gidx, dstx, zeros)
    xw2, lp2 = _proj_fused(part1, lp1, bases2, w_comp2, loop_w2, bias2)
    part2 = _sc_agg(xw2.reshape(R * N, D), gidx, dstx, zeros)
    h2, cat = _final(x, part2, lp2)
    return (x, h2, cat)
